# Initial kernel scaffold; baseline (speedup 1.0000x reference)
import jax
import jax.numpy as jnp
from jax.experimental import pallas as pl


def kernel(x, edge_index, W1, b1, W2, b2):
    raise NotImplementedError("write your pallas kernel here")


# speedup vs baseline: 11.0513x; 11.0513x over previous
import functools

import jax
import jax.numpy as jnp
from jax import lax
from jax.experimental import pallas as pl
from jax.experimental.pallas import tpu as pltpu
from jax.experimental.pallas import tpu_sc as plsc

NC = 2
NS = 16
NW = NC * NS
CHUNK = 128

_mesh = plsc.VectorSubcoreMesh(
    core_axis_name="c", subcore_axis_name="s", num_cores=NC, num_subcores=NS
)


def _deg_call(col2d, ones_src, zeros_np16, n_pad, cpw):
    rps = n_pad // NS

    @functools.partial(
        pl.kernel,
        out_type=jax.ShapeDtypeStruct((NC, n_pad, 16), jnp.float32),
        mesh=_mesh,
        compiler_params=pltpu.CompilerParams(use_tc_tiling_on_sc=False),
        scratch_types=[
            pltpu.VMEM((cpw, CHUNK), jnp.int32),
            pltpu.VMEM((CHUNK, 16), jnp.float32),
            pltpu.VMEM_SHARED((n_pad, 16), jnp.float32),
        ],
    )
    def deg_kernel(col_hbm, ones_hbm, zero_hbm, out_hbm, colv, onesb, acc):
        cid = lax.axis_index("c")
        sid = lax.axis_index("s")
        wid = cid * NS + sid
        pltpu.sync_copy(ones_hbm, onesb)
        pltpu.sync_copy(
            zero_hbm.at[pl.ds(sid * rps, rps)], acc.at[pl.ds(sid * rps, rps)]
        )
        pltpu.sync_copy(col_hbm.at[pl.ds(wid * cpw, cpw)], colv)
        plsc.subcore_barrier()

        @pl.loop(0, cpw)
        def _(j):
            pltpu.sync_copy(onesb, acc.at[colv.at[j]], add=True)

        plsc.subcore_barrier()
        pltpu.sync_copy(
            acc.at[pl.ds(sid * rps, rps)],
            out_hbm.at[cid, pl.ds(sid * rps, rps)],
        )

    return deg_kernel(col2d, ones_src, zeros_np16)


def _agg_call(y, row2d, col2d, zeros_npd, n_pad, cpw, d):
    rps = n_pad // NS

    @functools.partial(
        pl.kernel,
        out_type=jax.ShapeDtypeStruct((NC, n_pad, d), jnp.float32),
        mesh=_mesh,
        compiler_params=pltpu.CompilerParams(use_tc_tiling_on_sc=False),
        scratch_types=[
            pltpu.VMEM((cpw, CHUNK), jnp.int32),
            pltpu.VMEM((cpw, CHUNK), jnp.int32),
            pltpu.VMEM((CHUNK, d), jnp.float32),
            pltpu.VMEM_SHARED((n_pad, d), jnp.float32),
            pltpu.SemaphoreType.DMA,
        ],
    )
    def agg_kernel(y_hbm, row_hbm, col_hbm, zero_hbm, out_hbm,
                   rowv, colv, gbuf, acc, sem):
        cid = lax.axis_index("c")
        sid = lax.axis_index("s")
        wid = cid * NS + sid
        pltpu.sync_copy(
            zero_hbm.at[pl.ds(sid * rps, rps)], acc.at[pl.ds(sid * rps, rps)]
        )
        pltpu.sync_copy(row_hbm.at[pl.ds(wid * cpw, cpw)], rowv)
        pltpu.sync_copy(col_hbm.at[pl.ds(wid * cpw, cpw)], colv)
        plsc.subcore_barrier()

        @pl.loop(0, cpw)
        def _(j):
            pltpu.async_copy(y_hbm.at[rowv.at[j]], gbuf, sem).wait()
            pltpu.sync_copy(gbuf, acc.at[colv.at[j]], add=True)

        plsc.subcore_barrier()
        pltpu.sync_copy(
            acc.at[pl.ds(sid * rps, rps)],
            out_hbm.at[cid, pl.ds(sid * rps, rps)],
        )

    return agg_kernel(y, row2d, col2d, zeros_npd)


def _matmul1(x_pad, w1):
    def body(x_ref, w_ref, o_ref):
        o_ref[...] = jnp.dot(
            x_ref[...], w_ref[...],
            preferred_element_type=jnp.float32,
            precision=lax.Precision.HIGHEST,
        )

    return pl.pallas_call(
        body,
        out_shape=jax.ShapeDtypeStruct((x_pad.shape[0], w1.shape[1]), jnp.float32),
    )(x_pad, w1)


def _scale1(deg_parts, xw):
    def body(dp_ref, xw_ref, y_ref, dis_ref):
        deg = 1.0 + dp_ref[0, :, :1] + dp_ref[1, :, :1]
        dis = lax.rsqrt(deg)
        dis_ref[...] = dis
        y_ref[...] = dis * xw_ref[...]

    n_pad = xw.shape[0]
    return pl.pallas_call(
        body,
        out_shape=(
            jax.ShapeDtypeStruct((n_pad, xw.shape[1]), jnp.float32),
            jax.ShapeDtypeStruct((n_pad, 1), jnp.float32),
        ),
    )(deg_parts, xw)


def _layer2(p1, y1, dis, b1, w2p, n_real):
    def body(p_ref, y_ref, dis_ref, b_ref, w_ref, o_ref):
        dis = dis_ref[...]
        t = dis * (p_ref[0] + p_ref[1] + y_ref[...]) + b_ref[...]
        h = jnp.maximum(t, 0.0)
        rows = lax.broadcasted_iota(jnp.int32, h.shape, 0)
        h = jnp.where(rows < n_real, h, 0.0)
        hw = jnp.dot(
            h, w_ref[...],
            preferred_element_type=jnp.float32,
            precision=lax.Precision.HIGHEST,
        )
        o_ref[...] = dis * hw

    n_pad = y1.shape[0]
    return pl.pallas_call(
        body,
        out_shape=jax.ShapeDtypeStruct((n_pad, w2p.shape[1]), jnp.float32),
    )(p1, y1, dis, b1.reshape(1, -1), w2p)


def _final(p2, y2, dis, b2p, n_real, c_real):
    def body(p_ref, y_ref, dis_ref, b_ref, o_ref):
        z = dis_ref[...] * (p_ref[0] + p_ref[1] + y_ref[...]) + b_ref[...]
        cols = lax.broadcasted_iota(jnp.int32, z.shape, 1)
        mask = cols < c_real
        neg = jnp.float32(-1e30)
        zm = jnp.where(mask, z, neg)
        m = jnp.max(zm, axis=1, keepdims=True)
        s = jnp.sum(jnp.where(mask, jnp.exp(z - m), 0.0), axis=1, keepdims=True)
        res = z - m - jnp.log(s)
        o_ref[...] = res[:n_real, :c_real]

    return pl.pallas_call(
        body,
        out_shape=jax.ShapeDtypeStruct((n_real, c_real), jnp.float32),
    )(p2, y2, dis, b2p.reshape(1, -1))


def kernel(x, edge_index, W1, b1, W2, b2):
    n = x.shape[0]
    f_in = x.shape[1]
    hid = W1.shape[1]
    n_cls = W2.shape[1]
    e = edge_index.shape[1]

    n_pad = -(-(n + 1) // 128) * 128
    cpw = -(-e // (NW * CHUNK) // 8) * 8
    e_pad = cpw * NW * CHUNK
    d2 = -(-n_cls // 16) * 16

    row = edge_index[0].astype(jnp.int32)
    col = edge_index[1].astype(jnp.int32)
    pad_idx = jnp.full((e_pad - e,), n, jnp.int32)
    row2d = jnp.concatenate([row, pad_idx]).reshape(e_pad // CHUNK, CHUNK)
    col2d = jnp.concatenate([col, pad_idx]).reshape(e_pad // CHUNK, CHUNK)

    x_pad = jnp.zeros((n_pad, f_in), jnp.float32).at[:n].set(x)
    w2p = jnp.zeros((hid, d2), jnp.float32).at[:, :n_cls].set(W2)
    b2p = jnp.zeros((d2,), jnp.float32).at[:n_cls].set(b2)

    ones_src = jnp.ones((CHUNK, 16), jnp.float32)
    z16 = jnp.zeros((n_pad, 16), jnp.float32)
    zd1 = jnp.zeros((n_pad, hid), jnp.float32)
    zd2 = jnp.zeros((n_pad, d2), jnp.float32)

    deg_parts = _deg_call(col2d, ones_src, z16, n_pad, cpw)
    xw = _matmul1(x_pad, W1)
    y1, dis = _scale1(deg_parts, xw)
    p1 = _agg_call(y1, row2d, col2d, zd1, n_pad, cpw, hid)
    y2 = _layer2(p1, y1, dis, b1, w2p, n)
    p2 = _agg_call(y2, row2d, col2d, zd2, n_pad, cpw, d2)
    return _final(p2, y2, dis, b2p, n, n_cls)


# speedup vs baseline: 12.0634x; 1.0916x over previous
import functools

import jax
import jax.numpy as jnp
from jax import lax
from jax.experimental import pallas as pl
from jax.experimental.pallas import tpu as pltpu
from jax.experimental.pallas import tpu_sc as plsc

NC = 2
NS = 16
NW = NC * NS
CHUNK = 128

_mesh = plsc.VectorSubcoreMesh(
    core_axis_name="c", subcore_axis_name="s", num_cores=NC, num_subcores=NS
)


def _deg_call(col2d, ones_src, zeros_np16, n_pad, cpw):
    rps = n_pad // NS

    @functools.partial(
        pl.kernel,
        out_type=jax.ShapeDtypeStruct((NC, n_pad, 16), jnp.float32),
        mesh=_mesh,
        compiler_params=pltpu.CompilerParams(use_tc_tiling_on_sc=False),
        scratch_types=[
            pltpu.VMEM((cpw, CHUNK), jnp.int32),
            pltpu.VMEM((CHUNK, 16), jnp.float32),
            pltpu.VMEM_SHARED((n_pad, 16), jnp.float32),
        ],
    )
    def deg_kernel(col_hbm, ones_hbm, zero_hbm, out_hbm, colv, onesb, acc):
        cid = lax.axis_index("c")
        sid = lax.axis_index("s")
        wid = cid * NS + sid
        pltpu.sync_copy(ones_hbm, onesb)
        pltpu.sync_copy(
            zero_hbm.at[pl.ds(sid * rps, rps)], acc.at[pl.ds(sid * rps, rps)]
        )
        pltpu.sync_copy(col_hbm.at[pl.ds(wid * cpw, cpw)], colv)
        plsc.subcore_barrier()

        @pl.loop(0, cpw)
        def _(j):
            pltpu.sync_copy(onesb, acc.at[colv.at[j]], add=True)

        plsc.subcore_barrier()
        pltpu.sync_copy(
            acc.at[pl.ds(sid * rps, rps)],
            out_hbm.at[cid, pl.ds(sid * rps, rps)],
        )

    return deg_kernel(col2d, ones_src, zeros_np16)


def _agg_call(y, row2d, col2d, zeros_npd, n_pad, cpw, d):
    rps = n_pad // NS

    hc = cpw // 2

    @functools.partial(
        pl.kernel,
        out_type=jax.ShapeDtypeStruct((NC, n_pad, d), jnp.float32),
        mesh=_mesh,
        compiler_params=pltpu.CompilerParams(use_tc_tiling_on_sc=False),
        scratch_types=[
            pltpu.VMEM((hc, CHUNK), jnp.int32),
            pltpu.VMEM((hc, CHUNK), jnp.int32),
            pltpu.VMEM((CHUNK, d), jnp.float32),
            pltpu.VMEM((CHUNK, d), jnp.float32),
            pltpu.VMEM_SHARED((n_pad, d), jnp.float32),
            pltpu.SemaphoreType.DMA,
            pltpu.SemaphoreType.DMA,
            pltpu.SemaphoreType.DMA,
            pltpu.SemaphoreType.DMA,
        ],
    )
    def agg_kernel(y_hbm, row_hbm, col_hbm, zero_hbm, out_hbm,
                   rowv, colv, gb0, gb1, acc, sg0, sg1, ss0, ss1):
        cid = lax.axis_index("c")
        sid = lax.axis_index("s")
        wid = cid * NS + sid
        pltpu.sync_copy(
            zero_hbm.at[pl.ds(sid * rps, rps)], acc.at[pl.ds(sid * rps, rps)]
        )
        plsc.subcore_barrier()

        def do_phase(base):
            pltpu.sync_copy(row_hbm.at[pl.ds(base, hc)], rowv)
            pltpu.sync_copy(col_hbm.at[pl.ds(base, hc)], colv)
            pltpu.async_copy(y_hbm.at[rowv.at[0]], gb0, sg0)
            pltpu.async_copy(y_hbm.at[rowv.at[1]], gb1, sg1)

            @pl.loop(0, hc, step=2)
            def _(j):
                pltpu.make_async_copy(y_hbm.at[rowv.at[j]], gb0, sg0).wait()
                pltpu.async_copy(gb0, acc.at[colv.at[j]], ss0, add=True)
                pltpu.make_async_copy(y_hbm.at[rowv.at[j + 1]], gb1, sg1).wait()
                pltpu.async_copy(gb1, acc.at[colv.at[j + 1]], ss1, add=True)

                @pl.when(j + 2 < hc)
                def _():
                    pltpu.make_async_copy(gb0, acc.at[colv.at[j]], ss0).wait()
                    pltpu.async_copy(y_hbm.at[rowv.at[j + 2]], gb0, sg0)
                    pltpu.make_async_copy(gb1, acc.at[colv.at[j + 1]], ss1).wait()
                    pltpu.async_copy(y_hbm.at[rowv.at[j + 3]], gb1, sg1)

            pltpu.make_async_copy(gb0, acc.at[colv.at[hc - 2]], ss0).wait()
            pltpu.make_async_copy(gb1, acc.at[colv.at[hc - 1]], ss1).wait()

        do_phase(wid * cpw)
        do_phase(wid * cpw + hc)
        plsc.subcore_barrier()
        pltpu.sync_copy(
            acc.at[pl.ds(sid * rps, rps)],
            out_hbm.at[cid, pl.ds(sid * rps, rps)],
        )

    return agg_kernel(y, row2d, col2d, zeros_npd)


def _matmul1(x_pad, w1):
    def body(x_ref, w_ref, o_ref):
        o_ref[...] = jnp.dot(
            x_ref[...], w_ref[...],
            preferred_element_type=jnp.float32,
            precision=lax.Precision.HIGHEST,
        )

    return pl.pallas_call(
        body,
        out_shape=jax.ShapeDtypeStruct((x_pad.shape[0], w1.shape[1]), jnp.float32),
    )(x_pad, w1)


def _scale1(deg_parts, xw):
    def body(dp_ref, xw_ref, y_ref, dis_ref):
        deg = 1.0 + dp_ref[0, :, :1] + dp_ref[1, :, :1]
        dis = lax.rsqrt(deg)
        dis_ref[...] = dis
        y_ref[...] = dis * xw_ref[...]

    n_pad = xw.shape[0]
    return pl.pallas_call(
        body,
        out_shape=(
            jax.ShapeDtypeStruct((n_pad, xw.shape[1]), jnp.float32),
            jax.ShapeDtypeStruct((n_pad, 1), jnp.float32),
        ),
    )(deg_parts, xw)


def _layer2(p1, y1, dis, b1, w2p, n_real):
    def body(p_ref, y_ref, dis_ref, b_ref, w_ref, o_ref):
        dis = dis_ref[...]
        t = dis * (p_ref[0] + p_ref[1] + y_ref[...]) + b_ref[...]
        h = jnp.maximum(t, 0.0)
        rows = lax.broadcasted_iota(jnp.int32, h.shape, 0)
        h = jnp.where(rows < n_real, h, 0.0)
        hw = jnp.dot(
            h, w_ref[...],
            preferred_element_type=jnp.float32,
            precision=lax.Precision.HIGHEST,
        )
        o_ref[...] = dis * hw

    n_pad = y1.shape[0]
    return pl.pallas_call(
        body,
        out_shape=jax.ShapeDtypeStruct((n_pad, w2p.shape[1]), jnp.float32),
    )(p1, y1, dis, b1.reshape(1, -1), w2p)


def _final(p2, y2, dis, b2p, n_real, c_real):
    def body(p_ref, y_ref, dis_ref, b_ref, o_ref):
        z = dis_ref[...] * (p_ref[0] + p_ref[1] + y_ref[...]) + b_ref[...]
        cols = lax.broadcasted_iota(jnp.int32, z.shape, 1)
        mask = cols < c_real
        neg = jnp.float32(-1e30)
        zm = jnp.where(mask, z, neg)
        m = jnp.max(zm, axis=1, keepdims=True)
        s = jnp.sum(jnp.where(mask, jnp.exp(z - m), 0.0), axis=1, keepdims=True)
        res = z - m - jnp.log(s)
        o_ref[...] = res[:n_real, :c_real]

    return pl.pallas_call(
        body,
        out_shape=jax.ShapeDtypeStruct((n_real, c_real), jnp.float32),
    )(p2, y2, dis, b2p.reshape(1, -1))


def kernel(x, edge_index, W1, b1, W2, b2):
    n = x.shape[0]
    f_in = x.shape[1]
    hid = W1.shape[1]
    n_cls = W2.shape[1]
    e = edge_index.shape[1]

    n_pad = -(-(n + 1) // 128) * 128
    cpw = -(-e // (NW * CHUNK) // 8) * 8
    e_pad = cpw * NW * CHUNK
    d2 = -(-n_cls // 16) * 16

    row = edge_index[0].astype(jnp.int32)
    col = edge_index[1].astype(jnp.int32)
    pad_idx = jnp.full((e_pad - e,), n, jnp.int32)
    row2d = jnp.concatenate([row, pad_idx]).reshape(e_pad // CHUNK, CHUNK)
    col2d = jnp.concatenate([col, pad_idx]).reshape(e_pad // CHUNK, CHUNK)

    x_pad = jnp.zeros((n_pad, f_in), jnp.float32).at[:n].set(x)
    w2p = jnp.zeros((hid, d2), jnp.float32).at[:, :n_cls].set(W2)
    b2p = jnp.zeros((d2,), jnp.float32).at[:n_cls].set(b2)

    ones_src = jnp.ones((CHUNK, 16), jnp.float32)
    z16 = jnp.zeros((n_pad, 16), jnp.float32)
    zd1 = jnp.zeros((n_pad, hid), jnp.float32)
    zd2 = jnp.zeros((n_pad, d2), jnp.float32)

    deg_parts = _deg_call(col2d, ones_src, z16, n_pad, cpw)
    xw = _matmul1(x_pad, W1)
    y1, dis = _scale1(deg_parts, xw)
    p1 = _agg_call(y1, row2d, col2d, zd1, n_pad, cpw, hid)
    y2 = _layer2(p1, y1, dis, b1, w2p, n)
    p2 = _agg_call(y2, row2d, col2d, zd2, n_pad, cpw, d2)
    return _final(p2, y2, dis, b2p, n, n_cls)


# speedup vs baseline: 13.2729x; 1.1003x over previous
import functools

import jax
import jax.numpy as jnp
from jax import lax
from jax.experimental import pallas as pl
from jax.experimental.pallas import tpu as pltpu
from jax.experimental.pallas import tpu_sc as plsc

NC = 2
NS = 16
NW = NC * NS
CHUNK = 128

_mesh = plsc.VectorSubcoreMesh(
    core_axis_name="c", subcore_axis_name="s", num_cores=NC, num_subcores=NS
)


def _deg_call(col2d, ones_src, zeros_np16, n_pad, cpw):
    rps = n_pad // NS

    @functools.partial(
        pl.kernel,
        out_type=jax.ShapeDtypeStruct((NC, n_pad, 16), jnp.float32),
        mesh=_mesh,
        compiler_params=pltpu.CompilerParams(use_tc_tiling_on_sc=False),
        scratch_types=[
            pltpu.VMEM((cpw, CHUNK), jnp.int32),
            pltpu.VMEM((CHUNK, 16), jnp.float32),
            pltpu.VMEM_SHARED((n_pad, 16), jnp.float32),
        ],
    )
    def deg_kernel(col_hbm, ones_hbm, zero_hbm, out_hbm, colv, onesb, acc):
        cid = lax.axis_index("c")
        sid = lax.axis_index("s")
        wid = cid * NS + sid
        pltpu.sync_copy(ones_hbm, onesb)
        pltpu.sync_copy(
            zero_hbm.at[pl.ds(sid * rps, rps)], acc.at[pl.ds(sid * rps, rps)]
        )
        pltpu.sync_copy(col_hbm.at[pl.ds(wid * cpw, cpw)], colv)
        plsc.subcore_barrier()

        @pl.loop(0, cpw)
        def _(j):
            pltpu.sync_copy(onesb, acc.at[colv.at[j]], add=True)

        plsc.subcore_barrier()
        pltpu.sync_copy(
            acc.at[pl.ds(sid * rps, rps)],
            out_hbm.at[cid, pl.ds(sid * rps, rps)],
        )

    return deg_kernel(col2d, ones_src, zeros_np16)


def _agg_call(y, row2d, col2d, zeros_npd, n_pad, nch, d):
    rps = n_pad // NS
    ph = nch // 64

    @functools.partial(
        pl.kernel,
        out_type=jax.ShapeDtypeStruct((NC, n_pad, d), jnp.float32),
        mesh=_mesh,
        compiler_params=pltpu.CompilerParams(use_tc_tiling_on_sc=False),
        scratch_types=[
            pltpu.VMEM((ph, CHUNK), jnp.int32),
            pltpu.VMEM((ph, CHUNK), jnp.int32),
            pltpu.VMEM((CHUNK, d), jnp.float32),
            pltpu.VMEM((CHUNK, d), jnp.float32),
            pltpu.VMEM_SHARED((n_pad, d), jnp.float32),
            pltpu.SemaphoreType.DMA,
            pltpu.SemaphoreType.DMA,
            pltpu.SemaphoreType.DMA,
            pltpu.SemaphoreType.DMA,
        ],
    )
    def agg_kernel(y_hbm, row_hbm, col_hbm, zero_hbm, out_hbm,
                   rowv, colv, gb0, gb1, acc, sg0, sg1, ss0, ss1):
        cid = lax.axis_index("c")
        sid = lax.axis_index("s")
        wid = cid * NS + sid
        pltpu.sync_copy(
            zero_hbm.at[pl.ds(sid * rps, rps)], acc.at[pl.ds(sid * rps, rps)]
        )
        plsc.subcore_barrier()

        def do_phase(base):
            pltpu.sync_copy(row_hbm.at[pl.ds(base, ph)], rowv)
            pltpu.sync_copy(col_hbm.at[pl.ds(base, ph)], colv)
            pltpu.async_copy(y_hbm.at[rowv.at[0]], gb0, sg0)
            pltpu.async_copy(y_hbm.at[rowv.at[1]], gb1, sg1)

            @pl.loop(0, ph, step=2)
            def _(j):
                pltpu.make_async_copy(y_hbm.at[rowv.at[j]], gb0, sg0).wait()
                pltpu.async_copy(gb0, acc.at[colv.at[j]], ss0, add=True)
                pltpu.make_async_copy(y_hbm.at[rowv.at[j + 1]], gb1, sg1).wait()
                pltpu.async_copy(gb1, acc.at[colv.at[j + 1]], ss1, add=True)

                @pl.when(j + 2 < ph)
                def _():
                    pltpu.make_async_copy(gb0, acc.at[colv.at[j]], ss0).wait()
                    pltpu.async_copy(y_hbm.at[rowv.at[j + 2]], gb0, sg0)
                    pltpu.make_async_copy(gb1, acc.at[colv.at[j + 1]], ss1).wait()
                    pltpu.async_copy(y_hbm.at[rowv.at[j + 3]], gb1, sg1)

            pltpu.make_async_copy(gb0, acc.at[colv.at[ph - 2]], ss0).wait()
            pltpu.make_async_copy(gb1, acc.at[colv.at[ph - 1]], ss1).wait()

        nph = 3 - 2 * cid
        base0 = cid * (NS * 3 * ph) + sid * nph * ph

        @pl.loop(0, nph)
        def _(p):
            do_phase(base0 + p * ph)

        plsc.subcore_barrier()
        pltpu.sync_copy(
            acc.at[pl.ds(sid * rps, rps)],
            out_hbm.at[cid, pl.ds(sid * rps, rps)],
        )

    return agg_kernel(y, row2d, col2d, zeros_npd)


def _matmul1(x_pad, w1):
    def body(x_ref, w_ref, o_ref):
        o_ref[...] = jnp.dot(
            x_ref[...], w_ref[...],
            preferred_element_type=jnp.float32,
            precision=lax.Precision.HIGHEST,
        )

    return pl.pallas_call(
        body,
        out_shape=jax.ShapeDtypeStruct((x_pad.shape[0], w1.shape[1]), jnp.float32),
    )(x_pad, w1)


def _scale1(deg_parts, xw):
    def body(dp_ref, xw_ref, y_ref, dis_ref):
        deg = 1.0 + dp_ref[0, :, :1] + dp_ref[1, :, :1]
        dis = lax.rsqrt(deg)
        dis_ref[...] = dis
        y_ref[...] = dis * xw_ref[...]

    n_pad = xw.shape[0]
    return pl.pallas_call(
        body,
        out_shape=(
            jax.ShapeDtypeStruct((n_pad, xw.shape[1]), jnp.float32),
            jax.ShapeDtypeStruct((n_pad, 1), jnp.float32),
        ),
    )(deg_parts, xw)


def _layer2(p1, y1, dis, b1, w2p, n_real):
    def body(p_ref, y_ref, dis_ref, b_ref, w_ref, o_ref):
        dis = dis_ref[...]
        t = dis * (p_ref[0] + p_ref[1] + y_ref[...]) + b_ref[...]
        h = jnp.maximum(t, 0.0)
        rows = lax.broadcasted_iota(jnp.int32, h.shape, 0)
        h = jnp.where(rows < n_real, h, 0.0)
        hw = jnp.dot(
            h, w_ref[...],
            preferred_element_type=jnp.float32,
            precision=lax.Precision.HIGHEST,
        )
        o_ref[...] = dis * hw

    n_pad = y1.shape[0]
    return pl.pallas_call(
        body,
        out_shape=jax.ShapeDtypeStruct((n_pad, w2p.shape[1]), jnp.float32),
    )(p1, y1, dis, b1.reshape(1, -1), w2p)


def _final(p2, y2, dis, b2p, n_real, c_real):
    def body(p_ref, y_ref, dis_ref, b_ref, o_ref):
        z = dis_ref[...] * (p_ref[0] + p_ref[1] + y_ref[...]) + b_ref[...]
        cols = lax.broadcasted_iota(jnp.int32, z.shape, 1)
        mask = cols < c_real
        neg = jnp.float32(-1e30)
        zm = jnp.where(mask, z, neg)
        m = jnp.max(zm, axis=1, keepdims=True)
        s = jnp.sum(jnp.where(mask, jnp.exp(z - m), 0.0), axis=1, keepdims=True)
        res = z - m - jnp.log(s)
        o_ref[...] = res[:n_real, :c_real]

    return pl.pallas_call(
        body,
        out_shape=jax.ShapeDtypeStruct((n_real, c_real), jnp.float32),
    )(p2, y2, dis, b2p.reshape(1, -1))


def kernel(x, edge_index, W1, b1, W2, b2):
    n = x.shape[0]
    f_in = x.shape[1]
    hid = W1.shape[1]
    n_cls = W2.shape[1]
    e = edge_index.shape[1]

    n_pad = -(-(n + 1) // 128) * 128
    e_pad = -(-e // (64 * 8 * CHUNK)) * (64 * 8 * CHUNK)
    nch = e_pad // CHUNK
    cpw = nch // NW
    d2 = -(-n_cls // 16) * 16

    row = edge_index[0].astype(jnp.int32)
    col = edge_index[1].astype(jnp.int32)
    pad_idx = jnp.full((e_pad - e,), n, jnp.int32)
    row2d = jnp.concatenate([row, pad_idx]).reshape(e_pad // CHUNK, CHUNK)
    col2d = jnp.concatenate([col, pad_idx]).reshape(e_pad // CHUNK, CHUNK)

    x_pad = jnp.zeros((n_pad, f_in), jnp.float32).at[:n].set(x)
    w2p = jnp.zeros((hid, d2), jnp.float32).at[:, :n_cls].set(W2)
    b2p = jnp.zeros((d2,), jnp.float32).at[:n_cls].set(b2)

    ones_src = jnp.ones((CHUNK, 16), jnp.float32)
    z16 = jnp.zeros((n_pad, 16), jnp.float32)
    zd1 = jnp.zeros((n_pad, hid), jnp.float32)
    zd2 = jnp.zeros((n_pad, d2), jnp.float32)

    deg_parts = _deg_call(col2d, ones_src, z16, n_pad, cpw)
    xw = _matmul1(x_pad, W1)
    y1, dis = _scale1(deg_parts, xw)
    p1 = _agg_call(y1, row2d, col2d, zd1, n_pad, nch, hid)
    y2 = _layer2(p1, y1, dis, b1, w2p, n)
    p2 = _agg_call(y2, row2d, col2d, zd2, n_pad, nch, d2)
    return _final(p2, y2, dis, b2p, n, n_cls)
